# baseline (speedup 1.0000x reference)
import functools

import jax
import jax.numpy as jnp
from jax import lax
from jax.experimental import pallas as pl
from jax.experimental.pallas import tpu as pltpu
from jax.experimental.pallas import tpu_sc as plsc

_D = 512
_HEADS = 8
_N_SH = 1201
_N_S = 390
_N_H = 811
_E_SH = 80000
_E_SS = 20000
_E_HH = 40000
_B_PRESC = 1024

_NC = 2
_NS = 16
_L = 16


def _ceil_to(x, m):
    return (x + m - 1) // m * m


def _adj_cfg(n, e, ch):
    rows = _ceil_to(_ceil_to(n, _NS) // _NS, 8)
    npad = _ceil_to(n, 8)
    return dict(n=n, e=e, ch=ch, rows=rows, npad=npad, slab=rows * npad,
                e_half=e // _NC, n_chunks=(e // _NC) // ch)


_CFG_SH = _adj_cfg(_N_SH, _E_SH, 8000)
_CFG_HH = _adj_cfg(_N_H, _E_HH, 10000)
_CFG_SS = _adj_cfg(_N_S, _E_SS, 5000)
_UNROLL = 5


@functools.lru_cache(maxsize=None)
def _make_adj_all():
    cfgs = (_CFG_SH, _CFG_HH, _CFG_SS)
    maxslab = max(c["slab"] for c in cfgs)
    maxch = max(c["ch"] for c in cfgs)
    sched = [(g, k) for g, c in enumerate(cfgs) for k in range(c["n_chunks"])]
    mesh = plsc.VectorSubcoreMesh(core_axis_name="c", subcore_axis_name="s",
                                  num_cores=_NC, num_subcores=_NS)

    @functools.partial(
        pl.kernel,
        out_type=[jax.ShapeDtypeStruct((_NC, _NS * c["slab"]), jnp.float32)
                  for c in cfgs],
        mesh=mesh,
        scratch_types=[
            pltpu.VMEM((maxslab,), jnp.float32),
            pltpu.VMEM((maxch,), jnp.int32),
            pltpu.VMEM((maxch,), jnp.int32),
            pltpu.SemaphoreType.DMA,
            pltpu.SemaphoreType.DMA,
            pltpu.SemaphoreType.DMA,
        ],
        compiler_params=pltpu.CompilerParams(use_tc_tiling_on_sc=False,
                                             needs_layout_passes=False),
    )
    def adj(e_sh, e_hh, e_ss, zeros_hbm, out_sh, out_hh, out_ss,
            abuf, eb0, eb1, sem0, sem1, zsem):
        packs = (e_sh, e_hh, e_ss)
        outs = (out_sh, out_hh, out_ss)
        ebufs = (eb0, eb1)
        sems = (sem0, sem1)
        c = lax.axis_index("c")
        s = lax.axis_index("s")
        ones = jnp.full((_L,), 1.0, jnp.float32)

        def start(i):
            g, k = sched[i]
            cf = cfgs[g]
            off = c * cf["e_half"] + k * cf["ch"]
            b = i % 2
            return pltpu.async_copy(packs[g].at[pl.ds(off, cf["ch"])],
                                    ebufs[b].at[pl.ds(0, cf["ch"])], sems[b])

        zcp = pltpu.async_copy(zeros_hbm.at[pl.ds(0, _CFG_SH["slab"])], abuf, zsem)
        pending = [None, None]
        pending[0] = start(0)
        zcp.wait()

        i = 0
        for g, cf in enumerate(cfgs):
            slab = cf["slab"]
            losl = s * slab
            slab_u = jnp.uint32(slab)
            for _k in range(cf["n_chunks"]):
                b = i % 2
                if i + 1 < len(sched):
                    pending[(i + 1) % 2] = start(i + 1)
                pending[b].wait()
                ebuf = ebufs[b]

                def inner(ii, carry, ebuf=ebuf, losl=losl, slab_u=slab_u):
                    ib = ii * (_L * _UNROLL)
                    for j in range(_UNROLL):
                        pv = ebuf[pl.ds(ib + j * _L, _L)]
                        rf = pv - losl
                        m = lax.bitcast_convert_type(rf, jnp.uint32) < slab_u
                        plsc.addupdate_scatter(abuf, [rf], ones, mask=m)
                    return carry

                lax.fori_loop(0, cf["ch"] // (_L * _UNROLL), inner, 0)
                i += 1
            pltpu.sync_copy(abuf.at[pl.ds(0, slab)],
                            outs[g].at[c, pl.ds(s * slab, slab)])
            if g + 1 < len(cfgs):
                nslab = cfgs[g + 1]["slab"]
                pltpu.sync_copy(zeros_hbm.at[pl.ds(0, nslab)],
                                abuf.at[pl.ds(0, nslab)])

    return adj


def _adj_all(packed_sh, packed_hh, packed_ss, zeros):
    return _make_adj_all()(packed_sh, packed_hh, packed_ss, zeros)


def _prep_adj(ap, n):
    def body(ap_ref, an_ref, rm_ref):
        a = (ap_ref[0] + ap_ref[1])[:n, :n]
        cnt = jnp.sum(a, axis=1)
        inv = 1.0 / jnp.maximum(cnt, 1.0)
        an_ref[...] = a * inv[:, None]
        rm_ref[...] = (cnt > 0).astype(jnp.float32)[:, None]

    return pl.pallas_call(body, out_shape=[
        jax.ShapeDtypeStruct((n, n), jnp.float32),
        jax.ShapeDtypeStruct((n, 1), jnp.float32),
    ])(ap)


def _mk_helpers(P):
    f32 = jnp.float32

    def mmT(x, w):
        return lax.dot_general(x, w, (((1,), (1,)), ((), ())),
                               preferred_element_type=f32)

    def mm(a, b):
        return lax.dot_general(a, b, (((1,), (0,)), ((), ())),
                               preferred_element_type=f32)

    def lin(x, name):
        return mmT(x, P[name + "_w"]) + P[name + "_b"]

    def bn(x, name):
        m = jnp.mean(x, axis=0)
        xc = x - m
        v = jnp.mean(xc * xc, axis=0)
        return (xc / jnp.sqrt(v + 1e-5)) * P[name + "_g"] + P[name + "_be"]

    return mmT, mm, lin, bn


_S1_NAMES = ("SH_s_mlp", "SH_s_bn", "SH_h_mlp", "SH_h_bn", "SS_s_mlp",
             "SS_s_bn", "HH_h_mlp", "HH_h_bn", "kg_HH_mlp", "kg_HH_bn")


def _tc_stage1(sh, s, h, kg, pvals, pkeys):
    n_in = 4 + len(pvals)

    def body(*refs):
        sh_ref, s_ref, h_ref, kg_ref = refs[:4]
        prefs = refs[4:n_in]
        o_esh0, o_esh02, o_es0, o_ehkg = refs[n_in:]
        P = {k: prefs[i][...] for i, k in enumerate(pkeys)}
        _, _, lin, bn = _mk_helpers(P)
        tanh = jnp.tanh
        o_esh0[...] = tanh(bn(lin(sh_ref[...], "SH_s_mlp"), "SH_s_bn"))
        o_esh02[...] = tanh(bn(lin(sh_ref[...], "SH_h_mlp"), "SH_h_bn"))
        o_es0[...] = tanh(bn(lin(s_ref[...], "SS_s_mlp"), "SS_s_bn"))
        eh0 = tanh(bn(lin(h_ref[...], "HH_h_mlp"), "HH_h_bn"))
        kg0 = tanh(bn(lin(kg_ref[...], "kg_HH_mlp"), "kg_HH_bn"))
        o_ehkg[...] = eh0 + kg0

    out_shape = [
        jax.ShapeDtypeStruct((_N_SH, _D), jnp.float32),
        jax.ShapeDtypeStruct((_N_SH, _D), jnp.float32),
        jax.ShapeDtypeStruct((_N_S, _D), jnp.float32),
        jax.ShapeDtypeStruct((_N_H, _D), jnp.float32),
    ]
    return pl.pallas_call(body, out_shape=out_shape)(sh, s, h, kg, *pvals)


def _tc_forward(a_sh, rm_sh, a_ss, rm_ss, a_hh, rm_hh, esh0_in, esh02_in,
                es0_in, ehkg_in, presc, hm, hmt, pvals, pkeys):
    n_in = 13 + len(pvals)

    def body(*refs):
        (ash_ref, rmsh_ref, ass_ref, rmss_ref, ahh_ref, rmhh_ref,
         esh0_ref, esh02_ref, es0_ref, ehkg_ref, presc_ref,
         hm_ref, hmt_ref) = refs[:13]
        prefs = refs[13:n_in]
        o_es, o_eh, o_sy = refs[n_in:]
        P = {k: prefs[i][...] for i, k in enumerate(pkeys)}
        f32 = jnp.float32
        mmT, mm, lin, bn = _mk_helpers(P)
        tanh = jnp.tanh

        A_sh, inv_sh, rm_sh = ash_ref[...], None, rmsh_ref[...]
        A_ss, inv_ss, rm_ss = ass_ref[...], None, rmss_ref[...]
        A_hh, inv_hh, rm_hh = ahh_ref[...], None, rmhh_ref[...]

        def gcn(x, name, A, inv, rm):
            y = mmT(x, P[name + "_w"])
            return tanh(mm(A, y) + P[name + "_b"] * rm)

        def mha(q, kv1, kv2, pre):
            Q = lin(q, pre + "_WQ")
            K1 = lin(kv1, pre + "_WK")
            K2 = lin(kv2, pre + "_WK")
            V1 = lin(kv1, pre + "_WV")
            V2 = lin(kv2, pre + "_WV")
            hmv = hm_ref[...]
            hmtv = hmt_ref[...]
            sc = 1.0 / jnp.sqrt(f32(256 // _HEADS))
            x1 = jnp.exp(mm(Q * K1, hmv) * sc)
            x2 = jnp.exp(mm(Q * K2, hmv) * sc)
            den = 1.0 + x1 + x2
            ctx = mm(x1 / den, hmtv) * V1 + mm(x2 / den, hmtv) * V2
            return lin(ctx, pre + "_fc")

        esh0 = esh0_ref[...]
        esh02 = esh02_ref[...]
        es0 = es0_ref[...]
        eh0kg = ehkg_ref[...]

        b0 = gcn(esh0, "convSH1", A_sh, inv_sh, rm_sh)
        b1 = tanh(bn(lin(esh0 + b0, "SH_line_s_1"), "SH_bn_s_1"))
        b1N = gcn(b1, "convSH2", A_sh, inv_sh, rm_sh)
        b2_sh = tanh(bn(lin(b1 + b1N, "SH_line_s_2"), "SH_bn_s_2"))

        b0h = gcn(esh02, "convSH1h", A_sh, inv_sh, rm_sh)
        b1h = tanh(bn(lin(esh02 + b0h, "SH_line_h_1"), "SH_bn_h_1"))
        b1hN = gcn(b1h, "convSH2h", A_sh, inv_sh, rm_sh)
        b2_sh2 = tanh(bn(lin(b1h + b1hN, "SH_line_h_2"), "SH_bn_h_2"))

        r0 = gcn(es0, "convSS1", A_ss, inv_ss, rm_ss)
        r1s = tanh(bn(lin(es0 + r0, "SS_line_1"), "SS_bn_1"))
        r1N = gcn(r1s, "convSS2", A_ss, inv_ss, rm_ss)
        r2_s = tanh(bn(lin(r1s + r1N, "SS_line_2"), "SS_bn_2"))

        rh0 = gcn(eh0kg, "convHH1", A_hh, inv_hh, rm_hh)
        r1h = tanh(bn(lin(eh0kg + rh0, "HH_line_1"), "HH_bn_1"))
        r1hN = gcn(r1h, "convHH2", A_hh, inv_hh, rm_hh)
        r2_h = tanh(bn(lin(r1h + r1hN, "HH_line_2"), "HH_bn_2"))

        es = mha(b2_sh[:_N_S] + r2_s, b2_sh[:_N_S], r2_s, "att_s")
        es = tanh(bn(es, "es_bn_1"))
        ehx = mha(b2_sh2[_N_S:] + r2_h, b2_sh2[_N_S:], r2_h, "att_h")
        ehx = tanh(bn(ehx, "eh_bn_1"))

        o_es[...] = es
        o_eh[...] = ehx
        o_sy[...] = mm(presc_ref[...], es)

    out_shape = [
        jax.ShapeDtypeStruct((_N_S, 256), jnp.float32),
        jax.ShapeDtypeStruct((_N_H, 256), jnp.float32),
        jax.ShapeDtypeStruct((_B_PRESC, 256), jnp.float32),
    ]
    return pl.pallas_call(body, out_shape=out_shape)(
        a_sh, rm_sh, a_ss, rm_ss, a_hh, rm_hh, esh0_in, esh02_in,
        es0_in, ehkg_in, presc, hm, hmt, *pvals)


def kernel(sh_tensor, s_tensor, h_tensor, edge_index_SH, edge_index_SS,
           edge_index_HH, prescription, kgOneHot, p, params):
    f32 = jnp.float32
    sh = jnp.asarray(sh_tensor, f32)
    s = jnp.asarray(s_tensor, f32)
    h = jnp.asarray(h_tensor, f32)
    presc = jnp.asarray(prescription, f32)
    kg = jnp.asarray(kgOneHot, f32)

    def pack(ei, cfg):
        src = jnp.asarray(ei[0], jnp.int32)
        dst = jnp.asarray(ei[1], jnp.int32)
        return dst * cfg["npad"] + src

    zeros = jnp.zeros((_CFG_SH["slab"],), f32)
    p_sh, p_hh, p_ss = _adj_all(pack(edge_index_SH, _CFG_SH),
                                pack(edge_index_HH, _CFG_HH),
                                pack(edge_index_SS, _CFG_SS), zeros)

    def unflat(p, cfg):
        return p.reshape(_NC, _NS * cfg["rows"], cfg["npad"])

    a_sh, rm_sh = _prep_adj(unflat(p_sh, _CFG_SH), _N_SH)
    a_ss, rm_ss = _prep_adj(unflat(p_ss, _CFG_SS), _N_S)
    a_hh, rm_hh = _prep_adj(unflat(p_hh, _CFG_HH), _N_H)

    hm = jnp.repeat(jnp.eye(_HEADS, dtype=f32), 256 // _HEADS, axis=0)
    hmt = hm.T

    allkeys = tuple(sorted(params.keys()))
    s1keys = tuple(k for k in allkeys
                   if any(k.startswith(nm + "_") for nm in _S1_NAMES))
    s2keys = tuple(k for k in allkeys if k not in s1keys)
    s1vals = [jnp.asarray(params[k], f32) for k in s1keys]
    s2vals = [jnp.asarray(params[k], f32) for k in s2keys]

    esh0, esh02, es0, ehkg = _tc_stage1(sh, s, h, kg, s1vals, s1keys)

    es, ehx, e_synd = _tc_forward(a_sh, rm_sh, a_ss, rm_ss, a_hh, rm_hh,
                                  esh0, esh02, es0, ehkg, presc, hm, hmt,
                                  s2vals, s2keys)
    out = jnp.concatenate([es, ehx, e_synd], axis=0)
    return out * jnp.asarray(p, out.dtype)
